# Initial kernel scaffold; baseline (speedup 1.0000x reference)
import jax
import jax.numpy as jnp
from jax.experimental import pallas as pl


def kernel(x, mus, covs, cols):
    raise NotImplementedError("write your pallas kernel here")


# speedup vs baseline: 15.9263x; 15.9263x over previous
import functools

import jax
import jax.numpy as jnp
from jax.experimental import pallas as pl

N = 8192
G = 2048
D = 2
C = 3
K = 16
EPS = 1e-06

BN = 256


def _render_block(x_ref, mus_ref, covs_ref, cols_ref, out_ref):
    x = x_ref[...]
    mu = mus_ref[...]
    cv = covs_ref[...]
    cols = cols_ref[...]

    x0 = x[:, 0:1]
    x1 = x[:, 1:2]
    dx = x0 - mu[0:1, :]
    dy = x1 - mu[1:2, :]

    c00 = cv[0:1, :]
    c01 = cv[1:2, :]
    c10 = cv[2:3, :]
    c11 = cv[3:4, :]
    inv_det = 1.0 / (c00 * c11 - c01 * c10)
    quad = (c11 * dx * dx - (c01 + c10) * dx * dy + c00 * dy * dy) * inv_det
    gauss = jnp.exp(-0.5 * quad)

    iota = jax.lax.broadcasted_iota(jnp.int32, (BN, G), 1)
    g = gauss
    w = jnp.zeros((BN, G), jnp.float32)
    for _ in range(K):
        v = jnp.max(g, axis=1, keepdims=True)
        eq = g == v
        first = jnp.min(jnp.where(eq, iota, G), axis=1, keepdims=True)
        pos = iota == first
        w = jnp.where(pos, gauss, w)
        g = jnp.where(pos, -1.0, g)

    num = jnp.dot(w, cols, preferred_element_type=jnp.float32)
    den = jnp.sum(w, axis=1, keepdims=True) + EPS
    out_ref[...] = num / den


@jax.jit
def kernel(x, mus, covs, cols):
    mus_t = mus[0].T
    covs4 = covs[0].reshape(G, 4).T
    cols2 = cols[0]
    grid = (N // BN,)
    out = pl.pallas_call(
        _render_block,
        grid=grid,
        in_specs=[
            pl.BlockSpec((BN, D), lambda i: (i, 0)),
            pl.BlockSpec((D, G), lambda i: (0, 0)),
            pl.BlockSpec((4, G), lambda i: (0, 0)),
            pl.BlockSpec((G, C), lambda i: (0, 0)),
        ],
        out_specs=pl.BlockSpec((BN, C), lambda i: (i, 0)),
        out_shape=jax.ShapeDtypeStruct((N, C), jnp.float32),
    )(x, mus_t, covs4, cols2)
    return out
